# baseline (speedup 1.0000x reference)
import functools
import jax
import jax.numpy as jnp
from jax import lax
from jax.experimental import pallas as pl
from jax.experimental.pallas import tpu as pltpu
from jax.experimental.pallas import tpu_sc as plsc

NX = 768
NF = 768
NX8 = NX // 8
NBITS = 8

NC, NS = 2, 16
NW = NC * NS
GPW = NX8 // NW
ROWS_PW = GPW * 8
FV = NF // 16

_MSB = -(2 ** 31)


def _sc_decode_body(codes_hbm, scale_hbm, wqt_hbm, codes_v, scale_v, out_v):
    wid = lax.axis_index("s") * NC + lax.axis_index("c")
    g0 = wid * GPW
    pltpu.sync_copy(scale_hbm, scale_v)
    pltpu.sync_copy(codes_hbm.at[pl.ds(g0, GPW)], codes_v)

    def fv_body(fv, carry):
        fsl = pl.ds(fv * 16, 16)
        for gl in range(GPW):
            acc = [jnp.zeros((16,), jnp.float32) for _ in range(8)]
            for i in range(NBITS):
                v = codes_v[gl, i, fsl]
                sv = scale_v[i, fsl]
                nsv = -sv
                for p in range(8):
                    t = v << (24 + p)
                    acc[p] = acc[p] + jnp.where(t < 0, sv, nsv)
            for p in range(8):
                out_v[gl * 8 + p, fsl] = acc[p]
        return carry

    lax.fori_loop(0, FV, fv_body, 0)
    pltpu.sync_copy(out_v, wqt_hbm.at[pl.ds(wid * ROWS_PW, ROWS_PW)])


def _tc_matmul_body(x_ref, wqt_ref, bias_ref, out_ref):
    out = lax.dot_general(
        x_ref[...], wqt_ref[...], (((1,), (0,)), ((), ())),
        preferred_element_type=jnp.float32,
    )
    out_ref[...] = out + bias_ref[...]


def kernel(x, scale, bias, binary):
    size_out = x.shape[:-1] + (NF,)
    x2 = x.reshape(-1, NX)

    codes_t = binary.transpose(2, 0, 1)
    scale2 = scale.reshape(NBITS, NF)

    sc_decode = functools.partial(
        pl.kernel,
        out_type=jax.ShapeDtypeStruct((NX, NF), jnp.float32),
        mesh=plsc.VectorSubcoreMesh(
            core_axis_name="c", subcore_axis_name="s",
            num_cores=NC, num_subcores=NS,
        ),
        scratch_types=[
            pltpu.VMEM((GPW, NBITS, NF), jnp.int32),
            pltpu.VMEM((NBITS, NF), jnp.float32),
            pltpu.VMEM((ROWS_PW, NF), jnp.float32),
        ],
    )(_sc_decode_body)
    wqt = sc_decode(codes_t, scale2)

    out = pl.pallas_call(
        _tc_matmul_body,
        out_shape=jax.ShapeDtypeStruct((x2.shape[0], NF), jnp.float32),
    )(x2, wqt, bias.reshape(1, NF))
    return out.reshape(size_out)
